# Initial kernel scaffold; baseline (speedup 1.0000x reference)
import jax
import jax.numpy as jnp
from jax.experimental import pallas as pl


def kernel(x, table):
    raise NotImplementedError("write your pallas kernel here")


# speedup vs baseline: 1.9068x; 1.9068x over previous
import functools

import jax
import jax.numpy as jnp
from jax import lax
from jax.experimental import pallas as pl
from jax.experimental.pallas import tpu as pltpu
from jax.experimental.pallas import tpu_sc as plsc

B = 4096
L = 200
D = 32
NUM_EMB = 1_000_000

_info = plsc.get_sparse_core_info()
NC, NS, NL = _info.num_cores, _info.num_subcores, _info.num_lanes
NW = NC * NS
BPW = B // NW
IPW = BPW * L


def _make_kernel():
    mesh = plsc.VectorSubcoreMesh(core_axis_name="c", subcore_axis_name="s")

    @functools.partial(
        pl.kernel,
        mesh=mesh,
        out_type=jax.ShapeDtypeStruct((B, D), jnp.float32),
        compiler_params=pltpu.CompilerParams(use_tc_tiling_on_sc=False),
        scratch_types=[
            pltpu.VMEM((IPW,), jnp.int32),
            pltpu.VMEM((L, D), jnp.float32),
            pltpu.VMEM((BPW, D), jnp.float32),
            pltpu.SemaphoreType.DMA,
        ],
    )
    def emb_pool(x_hbm, table_hbm, out_hbm, idx_v, rows_v, out_v, sem):
        wid = lax.axis_index("s") * NC + lax.axis_index("c")
        pltpu.sync_copy(x_hbm.at[pl.ds(wid * IPW, IPW)], idx_v)

        def body(b, _):
            off = pl.multiple_of(b * L, 8)
            pltpu.async_copy(
                table_hbm.at[idx_v.at[pl.ds(off, L)]], rows_v, sem
            ).wait()

            def acc_body(l, acc):
                a0, a1 = acc
                return (a0 + rows_v[l, pl.ds(0, NL)],
                        a1 + rows_v[l, pl.ds(NL, NL)])

            z = jnp.zeros((NL,), jnp.float32)
            a0, a1 = lax.fori_loop(0, L, acc_body, (z, z))
            scale = jnp.float32(1.0 / L)
            out_v[b, pl.ds(0, NL)] = a0 * scale
            out_v[b, pl.ds(NL, NL)] = a1 * scale
            return _

        lax.fori_loop(0, BPW, body, 0)
        pltpu.sync_copy(out_v, out_hbm.at[pl.ds(wid * BPW, BPW)])

    return emb_pool


_emb_pool = _make_kernel()


@jax.jit
def kernel(x, table):
    return _emb_pool(x.reshape(-1), table)


# speedup vs baseline: 2.4485x; 1.2841x over previous
import functools

import jax
import jax.numpy as jnp
from jax import lax
from jax.experimental import pallas as pl
from jax.experimental.pallas import tpu as pltpu
from jax.experimental.pallas import tpu_sc as plsc

B = 4096
L = 200
D = 32
NUM_EMB = 1_000_000

_info = plsc.get_sparse_core_info()
NC, NS, NL = _info.num_cores, _info.num_subcores, _info.num_lanes
NW = NC * NS
BPW = B // NW
IPW = BPW * L
NBUF = 4
UNROLL = 8


def _make_kernel():
    mesh = plsc.VectorSubcoreMesh(core_axis_name="c", subcore_axis_name="s")

    @functools.partial(
        pl.kernel,
        mesh=mesh,
        out_type=jax.ShapeDtypeStruct((B, D), jnp.float32),
        compiler_params=pltpu.CompilerParams(use_tc_tiling_on_sc=False),
        scratch_types=[
            pltpu.VMEM((IPW,), jnp.int32),
            [pltpu.VMEM((L, D), jnp.float32) for _ in range(NBUF)],
            pltpu.VMEM((BPW, D), jnp.float32),
            [pltpu.SemaphoreType.DMA for _ in range(NBUF)],
        ],
    )
    def emb_pool(x_hbm, table_hbm, out_hbm, idx_v, bufs, out_v, sems):
        wid = lax.axis_index("s") * NC + lax.axis_index("c")
        pltpu.sync_copy(x_hbm.at[pl.ds(wid * IPW, IPW)], idx_v)

        def gather_desc(b, j):
            off = pl.multiple_of(b * L, 8)
            return pltpu.make_async_copy(
                table_hbm.at[idx_v.at[pl.ds(off, L)]], bufs[j], sems[j]
            )

        def process(b, j):
            gather_desc(b, j).wait()
            buf = bufs[j]

            def acc_body(k, accs):
                a0, a1, c0, c1 = accs
                base = k * UNROLL
                for u in range(UNROLL):
                    l = base + u
                    r0 = buf[l, pl.ds(0, NL)]
                    r1 = buf[l, pl.ds(NL, NL)]
                    if u % 2 == 0:
                        a0 = a0 + r0
                        a1 = a1 + r1
                    else:
                        c0 = c0 + r0
                        c1 = c1 + r1
                return a0, a1, c0, c1

            z = jnp.zeros((NL,), jnp.float32)
            a0, a1, c0, c1 = lax.fori_loop(0, L // UNROLL, acc_body,
                                           (z, z, z, z))
            scale = jnp.float32(1.0 / L)
            out_v[b, pl.ds(0, NL)] = (a0 + c0) * scale
            out_v[b, pl.ds(NL, NL)] = (a1 + c1) * scale

        for j in range(NBUF):
            gather_desc(j, j).start()

        def main_body(i, carry):
            for j in range(NBUF):
                b = i * NBUF + j
                process(b, j)
                gather_desc(b + NBUF, j).start()
            return carry

        lax.fori_loop(0, BPW // NBUF - 1, main_body, 0)

        for j in range(NBUF):
            process(BPW - NBUF + j, j)

        pltpu.sync_copy(out_v, out_hbm.at[pl.ds(wid * BPW, BPW)])

    return emb_pool


_emb_pool = _make_kernel()


@jax.jit
def kernel(x, table):
    return _emb_pool(x.reshape(-1), table)


# speedup vs baseline: 2.4828x; 1.0140x over previous
import functools

import jax
import jax.numpy as jnp
from jax import lax
from jax.experimental import pallas as pl
from jax.experimental.pallas import tpu as pltpu
from jax.experimental.pallas import tpu_sc as plsc

B = 4096
L = 200
D = 32
NUM_EMB = 1_000_000

_info = plsc.get_sparse_core_info()
NC, NS, NL = _info.num_cores, _info.num_subcores, _info.num_lanes
NW = NC * NS
BPW = B // NW
IPW = BPW * L
NBUF = 8
UNROLL = 8


def _make_kernel():
    mesh = plsc.VectorSubcoreMesh(core_axis_name="c", subcore_axis_name="s")

    @functools.partial(
        pl.kernel,
        mesh=mesh,
        out_type=jax.ShapeDtypeStruct((B, D), jnp.float32),
        compiler_params=pltpu.CompilerParams(use_tc_tiling_on_sc=False),
        scratch_types=[
            pltpu.VMEM((IPW,), jnp.int32),
            [pltpu.VMEM((L, D), jnp.float32) for _ in range(NBUF)],
            pltpu.VMEM((BPW, D), jnp.float32),
            [pltpu.SemaphoreType.DMA for _ in range(NBUF)],
        ],
    )
    def emb_pool(x_hbm, table_hbm, out_hbm, idx_v, bufs, out_v, sems):
        wid = lax.axis_index("s") * NC + lax.axis_index("c")
        pltpu.sync_copy(x_hbm.at[pl.ds(wid * IPW, IPW)], idx_v)

        def gather_desc(b, j):
            off = pl.multiple_of(b * L, 8)
            return pltpu.make_async_copy(
                table_hbm.at[idx_v.at[pl.ds(off, L)]], bufs[j], sems[j]
            )

        def process(b, j):
            gather_desc(b, j).wait()
            buf = bufs[j]

            def acc_body(k, accs):
                a0, a1, c0, c1 = accs
                base = k * UNROLL
                for u in range(UNROLL):
                    l = base + u
                    r0 = buf[l, pl.ds(0, NL)]
                    r1 = buf[l, pl.ds(NL, NL)]
                    if u % 2 == 0:
                        a0 = a0 + r0
                        a1 = a1 + r1
                    else:
                        c0 = c0 + r0
                        c1 = c1 + r1
                return a0, a1, c0, c1

            z = jnp.zeros((NL,), jnp.float32)
            a0, a1, c0, c1 = lax.fori_loop(0, L // UNROLL, acc_body,
                                           (z, z, z, z))
            scale = jnp.float32(1.0 / L)
            out_v[b, pl.ds(0, NL)] = (a0 + c0) * scale
            out_v[b, pl.ds(NL, NL)] = (a1 + c1) * scale

        for j in range(NBUF):
            gather_desc(j, j).start()

        def main_body(i, carry):
            for j in range(NBUF):
                b = i * NBUF + j
                process(b, j)
                gather_desc(b + NBUF, j).start()
            return carry

        lax.fori_loop(0, BPW // NBUF - 1, main_body, 0)

        for j in range(NBUF):
            process(BPW - NBUF + j, j)

        pltpu.sync_copy(out_v, out_hbm.at[pl.ds(wid * BPW, BPW)])

    return emb_pool


_emb_pool = _make_kernel()


@jax.jit
def kernel(x, table):
    return _emb_pool(x.reshape(-1), table)
